# baseline (speedup 1.0000x reference)
import functools

import jax
import jax.numpy as jnp
from jax import lax
from jax.experimental import pallas as pl
from jax.experimental.pallas import tpu as pltpu
from jax.experimental.pallas import tpu_sc as plsc

NC = 2
NS = 16
L = 16
NW = NC * NS


def _rsqrt_newton(d):
    bits = lax.bitcast_convert_type(d, jnp.int32)
    y = lax.bitcast_convert_type(
        jnp.int32(0x5F3759DF) - lax.shift_right_logical(bits, 1), jnp.float32)
    for _ in range(3):
        y = y * (1.5 - 0.5 * d * y * y)
    return y


def _zero_1d(ref, n):
    @plsc.parallel_loop(0, n // L, unroll=8)
    def z(i):
        ref[pl.ds(i * L, L)] = jnp.zeros((L,), jnp.float32)


def _read_flag(flag_hbm, fbuf):
    pltpu.sync_copy(flag_hbm, fbuf)
    return jnp.max(fbuf[pl.ds(0, L)])


def _make_norm_kernel(TOT, E_pad, NP, SH):
    EPTS = TOT // NS
    EPWS = TOT // NW
    EPT = E_pad // NS
    EPW = E_pad // NW
    SL = NP // NS
    MASK = (1 << SH) - 1
    mesh = plsc.VectorSubcoreMesh(
        core_axis_name="c", subcore_axis_name="s",
        num_cores=NC, num_subcores=NS)

    @functools.partial(
        pl.kernel, mesh=mesh,
        compiler_params=pltpu.CompilerParams(needs_layout_passes=False),
        out_type=(jax.ShapeDtypeStruct((TOT,), jnp.float32),
                  jax.ShapeDtypeStruct((E_pad,), jnp.float32)),
        scratch_types=[
            pltpu.VMEM((NP,), jnp.float32),
            pltpu.VMEM((NP,), jnp.float32),
            pltpu.VMEM((EPTS,), jnp.int32),
            pltpu.VMEM((EPTS,), jnp.float32),
            pltpu.VMEM((EPWS,), jnp.float32),
            pltpu.VMEM((SL,), jnp.float32),
            pltpu.VMEM((SL,), jnp.float32),
            pltpu.VMEM((128,), jnp.int32),
            pltpu.VMEM_SHARED((NS, NP), jnp.float32),
            pltpu.VMEM_SHARED((NP,), jnp.float32),
        ],
    )
    def norm_kernel(rcs_hbm, ws_hbm, rco_hbm, wb_hbm, flag_hbm,
                    norms_hbm, normb_hbm,
                    deg_l, dinv_l, rc_b, w_b, norm_b, tmp_b, acc_b, fbuf,
                    deg_sh, dinv_sh):
        c = lax.axis_index("c")
        s = lax.axis_index("s")
        wid = s * NC + c
        fs = _read_flag(flag_hbm, fbuf)

        _zero_1d(deg_l, NP)
        pltpu.sync_copy(rcs_hbm.at[pl.ds(s * EPTS, EPTS)], rc_b)
        pltpu.sync_copy(ws_hbm.at[pl.ds(s * EPTS, EPTS)], w_b)

        @plsc.parallel_loop(0, EPTS // L, unroll=8)
        def acc_deg(g):
            sl = pl.ds(g * L, L)
            r = lax.shift_right_logical(rc_b[sl], SH)
            plsc.addupdate_scatter(deg_l, [r], w_b[sl])

        @pl.when(fs > 0)
        def _():
            pltpu.sync_copy(rco_hbm.at[pl.ds(s * EPT, EPT)],
                            rc_b.at[pl.ds(0, EPT)])
            pltpu.sync_copy(wb_hbm.at[pl.ds(s * EPT, EPT)],
                            w_b.at[pl.ds(0, EPT)])

            @plsc.parallel_loop(0, EPT // L, unroll=8)
            def acc_deg_ovf(g):
                sl = pl.ds(g * L, L)
                r = lax.shift_right_logical(rc_b[sl], SH)
                plsc.addupdate_scatter(deg_l, [r], w_b[sl])

        pltpu.sync_copy(deg_l, deg_sh.at[s])
        plsc.subcore_barrier()

        base = s * SL
        _zero_1d(acc_b, SL)

        def red(j, _):
            pltpu.sync_copy(deg_sh.at[j, pl.ds(base, SL)], tmp_b)

            @plsc.parallel_loop(0, SL // L, unroll=8)
            def addg(g):
                sl = pl.ds(g * L, L)
                acc_b[sl] = acc_b[sl] + tmp_b[sl]
            return 0
        lax.fori_loop(0, NS, red, 0)

        @plsc.parallel_loop(0, SL // L, unroll=4)
        def din(g):
            sl = pl.ds(g * L, L)
            d = acc_b[sl]
            acc_b[sl] = jnp.where(d > 0.0, _rsqrt_newton(d), 0.0)

        pltpu.sync_copy(acc_b, dinv_sh.at[pl.ds(base, SL)])
        plsc.subcore_barrier()

        pltpu.sync_copy(dinv_sh, dinv_l)
        ebase = wid * EPWS
        pltpu.sync_copy(rcs_hbm.at[pl.ds(ebase, EPWS)],
                        rc_b.at[pl.ds(0, EPWS)])
        pltpu.sync_copy(ws_hbm.at[pl.ds(ebase, EPWS)],
                        w_b.at[pl.ds(0, EPWS)])

        @plsc.parallel_loop(0, EPWS // L, unroll=8)
        def nrm(g):
            sl = pl.ds(g * L, L)
            rc = rc_b[sl]
            dr = plsc.load_gather(dinv_l, [lax.shift_right_logical(rc, SH)])
            dc = plsc.load_gather(dinv_l, [rc & MASK])
            norm_b[sl] = (-w_b[sl]) * dr * dc

        pltpu.sync_copy(norm_b, norms_hbm.at[pl.ds(ebase, EPWS)])

        _zero_1d(norm_b, EPW)
        obase = wid * EPW

        @pl.when(fs > 0)
        def _():
            pltpu.sync_copy(rco_hbm.at[pl.ds(obase, EPW)],
                            rc_b.at[pl.ds(0, EPW)])
            pltpu.sync_copy(wb_hbm.at[pl.ds(obase, EPW)],
                            w_b.at[pl.ds(0, EPW)])

            @plsc.parallel_loop(0, EPW // L, unroll=8)
            def nrm_ovf(g):
                sl = pl.ds(g * L, L)
                rc = rc_b[sl]
                dr = plsc.load_gather(dinv_l,
                                      [lax.shift_right_logical(rc, SH)])
                dc = plsc.load_gather(dinv_l, [rc & MASK])
                norm_b[sl] = (-w_b[sl]) * dr * dc

        pltpu.sync_copy(norm_b.at[pl.ds(0, EPW)],
                        normb_hbm.at[pl.ds(obase, EPW)])

    return norm_kernel


def _make_prop_kernel(TOT, E_pad, NP, F, SH):
    FPW = F // NW
    FG = 4
    assert FPW % FG == 0
    CA = TOT // 32
    CB = E_pad // 32
    NCH = TOT // CA
    assert NCH % 2 == 0 and CB <= CA
    MASK = (1 << SH) - 1
    mesh = plsc.VectorSubcoreMesh(
        core_axis_name="c", subcore_axis_name="s",
        num_cores=NC, num_subcores=NS)

    @functools.partial(
        pl.kernel, mesh=mesh,
        compiler_params=pltpu.CompilerParams(needs_layout_passes=False),
        out_type=(jax.ShapeDtypeStruct((F, NP), jnp.float32),
                  jax.ShapeDtypeStruct((F, NP), jnp.float32)),
        scratch_types=(
            [pltpu.VMEM((NP,), jnp.float32) for _ in range(2 * FG)] +
            [pltpu.VMEM((CA,), jnp.int32) for _ in range(2)] +
            [pltpu.VMEM((CA,), jnp.float32) for _ in range(2)] +
            [pltpu.VMEM((128,), jnp.int32)] +
            [pltpu.SemaphoreType.DMA for _ in range(4)]
        ),
    )
    def prop_kernel(xT, rcs_hbm, norms_hbm, rco_hbm, normb_hbm, flag_hbm,
                    t1T, t2T,
                    a0, a1, a2, a3, b0, b1, b2, b3,
                    rc0, rc1, nb0, nb1, fbuf,
                    sr0, sr1, sn0, sn1):
        A = [a0, a1, a2, a3]
        B = [b0, b1, b2, b3]
        rcs, nbs = [rc0, rc1], [nb0, nb1]
        srs, sns = [sr0, sr1], [sn0, sn1]
        c = lax.axis_index("c")
        s = lax.axis_index("s")
        wid = s * NC + c
        f0 = wid * FPW
        fs = _read_flag(flag_hbm, fbuf)

        def start(ci, slot):
            eb = ci * CA
            pltpu.async_copy(rcs_hbm.at[pl.ds(eb, CA)], rcs[slot], srs[slot])
            pltpu.async_copy(norms_hbm.at[pl.ds(eb, CA)], nbs[slot],
                             sns[slot])

        def wait(slot):
            pltpu.make_async_copy(
                rcs_hbm.at[pl.ds(0, CA)], rcs[slot], srs[slot]).wait()
            pltpu.make_async_copy(
                norms_hbm.at[pl.ds(0, CA)], nbs[slot], sns[slot]).wait()

        def make_grp(rcb, nb, src, dst):
            def grp(g):
                sl = pl.ds(g * L, L)
                rc = rcb[sl]
                nv = nb[sl]
                cc = rc & MASK
                r = lax.shift_right_logical(rc, SH)
                vs = [plsc.load_gather(src[ff], [cc]) * nv
                      for ff in range(FG)]
                for ff in range(FG):
                    plsc.addupdate_scatter(dst[ff], [r], vs[ff])
            return grp

        def edge_sweep(src, dst):
            def process(slot):
                plsc.parallel_loop(0, CA // L, unroll=8)(
                    make_grp(rcs[slot], nbs[slot], src, dst))

            start(0, 0)

            def body2(ci2, _):
                ci = ci2 * 2
                start(ci + 1, 1)
                wait(0)
                process(0)

                @pl.when(ci2 < NCH // 2 - 1)
                def _():
                    start(ci + 2, 0)
                wait(1)
                process(1)
                return 0
            lax.fori_loop(0, NCH // 2, body2, 0)

            @pl.when(fs > 0)
            def _():
                def chunkb(ci, _):
                    eb = ci * CB
                    pltpu.sync_copy(rco_hbm.at[pl.ds(eb, CB)],
                                    rcs[0].at[pl.ds(0, CB)])
                    pltpu.sync_copy(normb_hbm.at[pl.ds(eb, CB)],
                                    nbs[0].at[pl.ds(0, CB)])
                    plsc.parallel_loop(0, CB // L, unroll=8)(
                        make_grp(rcs[0], nbs[0], src, dst))
                    return 0
                lax.fori_loop(0, E_pad // CB, chunkb, 0)

        for fg in range(FPW // FG):
            fbase = f0 + fg * FG
            for ff in range(FG):
                pltpu.sync_copy(xT.at[fbase + ff], A[ff])
                _zero_1d(B[ff], NP)
            edge_sweep(A, B)
            for ff in range(FG):
                pltpu.sync_copy(B[ff], t1T.at[fbase + ff])
            for ff in range(FG):
                _zero_1d(A[ff], NP)
            edge_sweep(B, A)
            for ff in range(FG):
                pltpu.sync_copy(xT.at[fbase + ff], B[ff])
                a, xt = A[ff], B[ff]

                @plsc.parallel_loop(0, NP // L, unroll=8)
                def comb(g):
                    sl = pl.ds(g * L, L)
                    a[sl] = 2.0 * a[sl] - xt[sl]

                pltpu.sync_copy(a, t2T.at[fbase + ff])

    return prop_kernel


def _matmul_relu(xT, t1T, t2T, W, b, N, NP, F_IN, F_OUT, BN=1024):
    def mm_kernel(x_ref, t1_ref, t2_ref, w0, w1, w2, b_ref, o_ref):
        dn = (((0,), (0,)), ((), ()))
        acc = lax.dot_general(x_ref[...], w0[...], dn,
                              preferred_element_type=jnp.float32)
        acc = acc + lax.dot_general(t1_ref[...], w1[...], dn,
                                    preferred_element_type=jnp.float32)
        acc = acc + lax.dot_general(t2_ref[...], w2[...], dn,
                                    preferred_element_type=jnp.float32)
        acc = acc + b_ref[...]
        o_ref[...] = jnp.maximum(acc, 0.0)

    grid = (pl.cdiv(N, BN),)
    fspec = pl.BlockSpec((F_IN, BN), lambda i: (0, i))
    wspec = pl.BlockSpec((F_IN, F_OUT), lambda i: (0, 0))
    return pl.pallas_call(
        mm_kernel,
        grid=grid,
        in_specs=[fspec, fspec, fspec, wspec, wspec, wspec,
                  pl.BlockSpec((1, F_OUT), lambda i: (0, 0))],
        out_specs=pl.BlockSpec((BN, F_OUT), lambda i: (i, 0)),
        out_shape=jax.ShapeDtypeStruct((N, F_OUT), jnp.float32),
    )(xT, t1T, t2T, W[0], W[1], W[2], b.reshape(1, F_OUT))


def kernel(x, edge_index, edge_weight, W, b):
    N, F_IN = x.shape
    F_OUT = W.shape[2]
    E = edge_weight.shape[0]
    i32 = jnp.int32

    E_pad = -(-E // (NW * 128)) * (NW * 128)
    pad = E_pad - E
    SH = max((N - 1).bit_length(), 4)
    assert 2 * SH <= 31
    lanes_pad = (jnp.arange(pad, dtype=i32) % L) * ((1 << SH) + 1)
    rc = jnp.concatenate(
        [edge_index[0] << SH | edge_index[1], lanes_pad])
    w_p = jnp.pad(edge_weight, (0, pad))
    NP = -(-N // (NS * L)) * (NS * L)

    B_pad = E_pad // L + 1024
    TOT = L * B_pad
    bank = lax.shift_right_logical(rc, SH) & (L - 1)
    oh = (bank[:, None] == jnp.arange(L, dtype=i32)[None, :]).astype(i32)
    rank = jnp.sum(jnp.cumsum(oh, axis=0) * oh, axis=1) - 1
    ovf = rank >= B_pad
    dest = jnp.where(ovf, TOT, rank * L + bank)
    lane = jnp.arange(TOT, dtype=i32) & (L - 1)
    rc_s = (lane * ((1 << SH) + 1)).at[dest].set(
        rc, mode="drop", unique_indices=True)
    w_s = jnp.zeros((TOT,), jnp.float32).at[dest].set(
        w_p, mode="drop", unique_indices=True)
    w_ovf = jnp.where(ovf, w_p, 0.0)
    flag = jnp.zeros((128,), i32) + jnp.any(ovf).astype(i32)

    norm_s, norm_ovf = _make_norm_kernel(TOT, E_pad, NP, SH)(
        rc_s, w_s, rc, w_ovf, flag)
    xTp = jnp.pad(x.T, ((0, 0), (0, NP - N)))
    t1T, t2T = _make_prop_kernel(TOT, E_pad, NP, F_IN, SH)(
        xTp, rc_s, norm_s, rc, norm_ovf, flag)
    return _matmul_relu(xTp, t1T, t2T, W, b, N, NP, F_IN, F_OUT)


# speedup vs baseline: 1.8360x; 1.8360x over previous
import functools

import jax
import jax.numpy as jnp
from jax import lax
from jax.experimental import pallas as pl
from jax.experimental.pallas import tpu as pltpu
from jax.experimental.pallas import tpu_sc as plsc

NC = 2
NS = 16
L = 16
NW = NC * NS
BL = 512


def _rsqrt_newton(d):
    bits = lax.bitcast_convert_type(d, jnp.int32)
    y = lax.bitcast_convert_type(
        jnp.int32(0x5F3759DF) - lax.shift_right_logical(bits, 1), jnp.float32)
    for _ in range(3):
        y = y * (1.5 - 0.5 * d * y * y)
    return y


def _zero_1d(ref, n):
    @plsc.parallel_loop(0, n // L, unroll=8)
    def z(i):
        ref[pl.ds(i * L, L)] = jnp.zeros((L,), jnp.float32)


def _make_norm_kernel(E_pad, NP, SH):
    EPT = E_pad // NS
    EPW = E_pad // NW
    SLOTW = BL * L
    SL = NP // NS
    MASK = (1 << SH) - 1
    mesh = plsc.VectorSubcoreMesh(
        core_axis_name="c", subcore_axis_name="s",
        num_cores=NC, num_subcores=NS)

    @functools.partial(
        pl.kernel, mesh=mesh,
        compiler_params=pltpu.CompilerParams(needs_layout_passes=False),
        out_type=(jax.ShapeDtypeStruct((NW * SLOTW,), jnp.int32),
                  jax.ShapeDtypeStruct((NW * SLOTW,), jnp.float32),
                  jax.ShapeDtypeStruct((E_pad,), jnp.float32)),
        scratch_types=[
            pltpu.VMEM((NP,), jnp.float32),
            pltpu.VMEM((NP,), jnp.float32),
            pltpu.VMEM((EPT,), jnp.int32),
            pltpu.VMEM((EPT,), jnp.float32),
            pltpu.VMEM((EPW,), jnp.int32),
            pltpu.VMEM((SLOTW,), jnp.int32),
            pltpu.VMEM((SLOTW,), jnp.float32),
            pltpu.VMEM((SL,), jnp.float32),
            pltpu.VMEM((SL,), jnp.float32),
            pltpu.VMEM_SHARED((NS, NP), jnp.float32),
            pltpu.VMEM_SHARED((NP,), jnp.float32),
        ],
    )
    def norm_kernel(rc_hbm, w_hbm, dl_hbm,
                    rcs_hbm, norms_hbm, normb_hbm,
                    deg_l, dinv_l, rc_b, w_b, dl_b, rcs_loc, norm_loc,
                    tmp_b, acc_b, deg_sh, dinv_sh):
        c = lax.axis_index("c")
        s = lax.axis_index("s")
        wid = s * NC + c

        _zero_1d(deg_l, NP)
        pltpu.sync_copy(rc_hbm.at[pl.ds(s * EPT, EPT)], rc_b)
        pltpu.sync_copy(w_hbm.at[pl.ds(s * EPT, EPT)], w_b)

        @plsc.parallel_loop(0, EPT // L, unroll=8)
        def acc_deg(g):
            sl = pl.ds(g * L, L)
            r = lax.shift_right_logical(rc_b[sl], SH)
            plsc.addupdate_scatter(deg_l, [r], w_b[sl])

        pltpu.sync_copy(deg_l, deg_sh.at[s])
        plsc.subcore_barrier()

        base = s * SL
        _zero_1d(acc_b, SL)

        def red(j, _):
            pltpu.sync_copy(deg_sh.at[j, pl.ds(base, SL)], tmp_b)

            @plsc.parallel_loop(0, SL // L, unroll=8)
            def addg(g):
                sl = pl.ds(g * L, L)
                acc_b[sl] = acc_b[sl] + tmp_b[sl]
            return 0
        lax.fori_loop(0, NS, red, 0)

        @plsc.parallel_loop(0, SL // L, unroll=4)
        def din(g):
            sl = pl.ds(g * L, L)
            d = acc_b[sl]
            acc_b[sl] = jnp.where(d > 0.0, _rsqrt_newton(d), 0.0)

        pltpu.sync_copy(acc_b, dinv_sh.at[pl.ds(base, SL)])
        plsc.subcore_barrier()

        pltpu.sync_copy(dinv_sh, dinv_l)
        ebase = wid * EPW
        pltpu.sync_copy(rc_hbm.at[pl.ds(ebase, EPW)], rc_b.at[pl.ds(0, EPW)])
        pltpu.sync_copy(w_hbm.at[pl.ds(ebase, EPW)], w_b.at[pl.ds(0, EPW)])
        pltpu.sync_copy(dl_hbm.at[pl.ds(ebase, EPW)], dl_b)

        patt = lax.iota(jnp.int32, L) * ((1 << SH) + 1)

        @plsc.parallel_loop(0, SLOTW // L, unroll=8)
        def initslots(i):
            sl = pl.ds(i * L, L)
            rcs_loc[sl] = patt
            norm_loc[sl] = jnp.zeros((L,), jnp.float32)

        @plsc.parallel_loop(0, EPW // L, unroll=8)
        def nrm(g):
            sl = pl.ds(g * L, L)
            rc = rc_b[sl]
            d = dl_b[sl]
            dr = plsc.load_gather(dinv_l, [lax.shift_right_logical(rc, SH)])
            dc = plsc.load_gather(dinv_l, [rc & MASK])
            nv = (-w_b[sl]) * dr * dc
            m = d < SLOTW
            plsc.store_scatter(rcs_loc, [d], rc, mask=m)
            plsc.store_scatter(norm_loc, [d], nv, mask=m)
            w_b[sl] = jnp.where(m, 0.0, nv)

        pltpu.sync_copy(rcs_loc, rcs_hbm.at[pl.ds(wid * SLOTW, SLOTW)])
        pltpu.sync_copy(norm_loc, norms_hbm.at[pl.ds(wid * SLOTW, SLOTW)])
        pltpu.sync_copy(w_b.at[pl.ds(0, EPW)],
                        normb_hbm.at[pl.ds(ebase, EPW)])

    return norm_kernel


def _make_prop_kernel(TOT, E_pad, NP, F, SH):
    FPW = F // NW
    FG = 4
    assert FPW % FG == 0
    CA = 4096
    CB = 4096
    NCH = TOT // CA
    assert TOT % CA == 0 and NCH % 2 == 0
    assert E_pad % CB == 0 and CB <= CA
    MASK = (1 << SH) - 1
    mesh = plsc.VectorSubcoreMesh(
        core_axis_name="c", subcore_axis_name="s",
        num_cores=NC, num_subcores=NS)

    @functools.partial(
        pl.kernel, mesh=mesh,
        compiler_params=pltpu.CompilerParams(needs_layout_passes=False),
        out_type=(jax.ShapeDtypeStruct((F, NP), jnp.float32),
                  jax.ShapeDtypeStruct((F, NP), jnp.float32)),
        scratch_types=(
            [pltpu.VMEM((NP,), jnp.float32) for _ in range(2 * FG)] +
            [pltpu.VMEM((CA,), jnp.int32) for _ in range(2)] +
            [pltpu.VMEM((CA,), jnp.float32) for _ in range(2)] +
            [pltpu.VMEM((128,), jnp.int32)] +
            [pltpu.SemaphoreType.DMA for _ in range(4)]
        ),
    )
    def prop_kernel(xT, rcs_hbm, norms_hbm, rco_hbm, normb_hbm, flag_hbm,
                    t1T, t2T,
                    a0, a1, a2, a3, b0, b1, b2, b3,
                    rc0, rc1, nb0, nb1, fbuf,
                    sr0, sr1, sn0, sn1):
        A = [a0, a1, a2, a3]
        B = [b0, b1, b2, b3]
        rcs, nbs = [rc0, rc1], [nb0, nb1]
        srs, sns = [sr0, sr1], [sn0, sn1]
        c = lax.axis_index("c")
        s = lax.axis_index("s")
        wid = s * NC + c
        f0 = wid * FPW
        pltpu.sync_copy(flag_hbm, fbuf)
        fs = jnp.max(fbuf[pl.ds(0, L)])

        def start(ci, slot):
            eb = ci * CA
            pltpu.async_copy(rcs_hbm.at[pl.ds(eb, CA)], rcs[slot], srs[slot])
            pltpu.async_copy(norms_hbm.at[pl.ds(eb, CA)], nbs[slot],
                             sns[slot])

        def wait(slot):
            pltpu.make_async_copy(
                rcs_hbm.at[pl.ds(0, CA)], rcs[slot], srs[slot]).wait()
            pltpu.make_async_copy(
                norms_hbm.at[pl.ds(0, CA)], nbs[slot], sns[slot]).wait()

        def make_grp(rcb, nb, src, dst):
            def grp(g):
                sl = pl.ds(g * L, L)
                rc = rcb[sl]
                nv = nb[sl]
                cc = rc & MASK
                r = lax.shift_right_logical(rc, SH)
                vs = [plsc.load_gather(src[ff], [cc]) * nv
                      for ff in range(FG)]
                for ff in range(FG):
                    plsc.addupdate_scatter(dst[ff], [r], vs[ff])
            return grp

        def edge_sweep(src, dst):
            def process(slot):
                plsc.parallel_loop(0, CA // L, unroll=8)(
                    make_grp(rcs[slot], nbs[slot], src, dst))

            start(0, 0)

            def body2(ci2, _):
                ci = ci2 * 2
                start(ci + 1, 1)
                wait(0)
                process(0)

                @pl.when(ci2 < NCH // 2 - 1)
                def _():
                    start(ci + 2, 0)
                wait(1)
                process(1)
                return 0
            lax.fori_loop(0, NCH // 2, body2, 0)

            @pl.when(fs > 0)
            def _():
                def chunkb(ci, _):
                    eb = ci * CB
                    pltpu.sync_copy(rco_hbm.at[pl.ds(eb, CB)],
                                    rcs[0].at[pl.ds(0, CB)])
                    pltpu.sync_copy(normb_hbm.at[pl.ds(eb, CB)],
                                    nbs[0].at[pl.ds(0, CB)])
                    plsc.parallel_loop(0, CB // L, unroll=8)(
                        make_grp(rcs[0], nbs[0], src, dst))
                    return 0
                lax.fori_loop(0, E_pad // CB, chunkb, 0)

        for fg in range(FPW // FG):
            fbase = f0 + fg * FG
            for ff in range(FG):
                pltpu.sync_copy(xT.at[fbase + ff], A[ff])
                _zero_1d(B[ff], NP)
            edge_sweep(A, B)
            for ff in range(FG):
                pltpu.sync_copy(B[ff], t1T.at[fbase + ff])
            for ff in range(FG):
                _zero_1d(A[ff], NP)
            edge_sweep(B, A)
            for ff in range(FG):
                pltpu.sync_copy(xT.at[fbase + ff], B[ff])
                a, xt = A[ff], B[ff]

                @plsc.parallel_loop(0, NP // L, unroll=8)
                def comb(g):
                    sl = pl.ds(g * L, L)
                    a[sl] = 2.0 * a[sl] - xt[sl]

                pltpu.sync_copy(a, t2T.at[fbase + ff])

    return prop_kernel


def _matmul_relu(xT, t1T, t2T, W, b, N, NP, F_IN, F_OUT, BN=1024):
    def mm_kernel(x_ref, t1_ref, t2_ref, w0, w1, w2, b_ref, o_ref):
        dn = (((0,), (0,)), ((), ()))
        acc = lax.dot_general(x_ref[...], w0[...], dn,
                              preferred_element_type=jnp.float32)
        acc = acc + lax.dot_general(t1_ref[...], w1[...], dn,
                                    preferred_element_type=jnp.float32)
        acc = acc + lax.dot_general(t2_ref[...], w2[...], dn,
                                    preferred_element_type=jnp.float32)
        acc = acc + b_ref[...]
        o_ref[...] = jnp.maximum(acc, 0.0)

    grid = (pl.cdiv(N, BN),)
    fspec = pl.BlockSpec((F_IN, BN), lambda i: (0, i))
    wspec = pl.BlockSpec((F_IN, F_OUT), lambda i: (0, 0))
    return pl.pallas_call(
        mm_kernel,
        grid=grid,
        in_specs=[fspec, fspec, fspec, wspec, wspec, wspec,
                  pl.BlockSpec((1, F_OUT), lambda i: (0, 0))],
        out_specs=pl.BlockSpec((BN, F_OUT), lambda i: (i, 0)),
        out_shape=jax.ShapeDtypeStruct((N, F_OUT), jnp.float32),
    )(xT, t1T, t2T, W[0], W[1], W[2], b.reshape(1, F_OUT))


def kernel(x, edge_index, edge_weight, W, b):
    N, F_IN = x.shape
    F_OUT = W.shape[2]
    E = edge_weight.shape[0]
    i32 = jnp.int32

    E_pad = -(-E // (NW * 128)) * (NW * 128)
    pad = E_pad - E
    SH = max((N - 1).bit_length(), 4)
    assert 2 * SH <= 31
    lanes_pad = (jnp.arange(pad, dtype=i32) % L) * ((1 << SH) + 1)
    rc = jnp.concatenate(
        [edge_index[0] << SH | edge_index[1], lanes_pad])
    w_p = jnp.pad(edge_weight, (0, pad))
    NP = -(-N // (NS * L)) * (NS * L)

    RPW = E_pad // NW
    SLOTW = BL * L
    TOT = NW * SLOTW
    bank = lax.shift_right_logical(rc, SH) & (L - 1)
    oh = (bank[:, None] == jnp.arange(L, dtype=i32)[None, :]).astype(i32)
    ohr = oh.reshape(NW, RPW, L)
    rank = (jnp.sum(jnp.cumsum(ohr, axis=1) * ohr, axis=-1) - 1).reshape(
        E_pad)
    ovf = rank >= BL
    dl = jnp.where(ovf, jnp.int32(1 << 20), rank * L + bank)
    flag = jnp.zeros((128,), i32) + jnp.any(ovf).astype(i32)

    rc_s, norm_s, norm_ovf = _make_norm_kernel(E_pad, NP, SH)(rc, w_p, dl)
    xTp = jnp.pad(x.T, ((0, 0), (0, NP - N)))
    t1T, t2T = _make_prop_kernel(TOT, E_pad, NP, F_IN, SH)(
        xTp, rc_s, norm_s, rc, norm_ovf, flag)
    return _matmul_relu(xTp, t1T, t2T, W, b, N, NP, F_IN, F_OUT)


# speedup vs baseline: 2.9341x; 1.5981x over previous
import functools

import jax
import jax.numpy as jnp
from jax import lax
from jax.experimental import pallas as pl
from jax.experimental.pallas import tpu as pltpu
from jax.experimental.pallas import tpu_sc as plsc

NC = 2
NS = 16
L = 16
NW = NC * NS
BL = 512


def _rsqrt_newton(d):
    bits = lax.bitcast_convert_type(d, jnp.int32)
    y = lax.bitcast_convert_type(
        jnp.int32(0x5F3759DF) - lax.shift_right_logical(bits, 1), jnp.float32)
    for _ in range(3):
        y = y * (1.5 - 0.5 * d * y * y)
    return y


def _zero_1d(ref, n):
    @plsc.parallel_loop(0, n // L, unroll=8)
    def z(i):
        ref[pl.ds(i * L, L)] = jnp.zeros((L,), jnp.float32)


def _make_norm_kernel(E_pad, NP, SH):
    EPT = E_pad // NS
    EPW = E_pad // NW
    SLOTW = BL * L
    SL = NP // NS
    MASK = (1 << SH) - 1
    mesh = plsc.VectorSubcoreMesh(
        core_axis_name="c", subcore_axis_name="s",
        num_cores=NC, num_subcores=NS)

    @functools.partial(
        pl.kernel, mesh=mesh,
        compiler_params=pltpu.CompilerParams(needs_layout_passes=False),
        out_type=(jax.ShapeDtypeStruct((NW * SLOTW,), jnp.int32),
                  jax.ShapeDtypeStruct((NW * SLOTW,), jnp.float32),
                  jax.ShapeDtypeStruct((E_pad,), jnp.float32),
                  jax.ShapeDtypeStruct((NW * L,), jnp.int32)),
        scratch_types=[
            pltpu.VMEM((NP,), jnp.float32),
            pltpu.VMEM((NP,), jnp.float32),
            pltpu.VMEM((EPT,), jnp.int32),
            pltpu.VMEM((EPT,), jnp.float32),
            pltpu.VMEM((128,), jnp.int32),
            pltpu.VMEM((SLOTW,), jnp.int32),
            pltpu.VMEM((SLOTW,), jnp.float32),
            pltpu.VMEM((SL,), jnp.float32),
            pltpu.VMEM((SL,), jnp.float32),
            pltpu.VMEM_SHARED((NS, NP), jnp.float32),
            pltpu.VMEM_SHARED((NP,), jnp.float32),
        ],
    )
    def norm_kernel(rc_hbm, w_hbm,
                    rcs_hbm, norms_hbm, normb_hbm, flag_hbm,
                    deg_l, dinv_l, rc_b, w_b, cnt_b, rcs_loc, norm_loc,
                    tmp_b, acc_b, deg_sh, dinv_sh):
        c = lax.axis_index("c")
        s = lax.axis_index("s")
        wid = s * NC + c

        _zero_1d(deg_l, NP)
        pltpu.sync_copy(rc_hbm.at[pl.ds(s * EPT, EPT)], rc_b)
        pltpu.sync_copy(w_hbm.at[pl.ds(s * EPT, EPT)], w_b)

        @plsc.parallel_loop(0, EPT // L, unroll=8)
        def acc_deg(g):
            sl = pl.ds(g * L, L)
            r = lax.shift_right_logical(rc_b[sl], SH)
            plsc.addupdate_scatter(deg_l, [r], w_b[sl])

        pltpu.sync_copy(deg_l, deg_sh.at[s])
        plsc.subcore_barrier()

        base = s * SL
        _zero_1d(acc_b, SL)

        def red(j, _):
            pltpu.sync_copy(deg_sh.at[j, pl.ds(base, SL)], tmp_b)

            @plsc.parallel_loop(0, SL // L, unroll=8)
            def addg(g):
                sl = pl.ds(g * L, L)
                acc_b[sl] = acc_b[sl] + tmp_b[sl]
            return 0
        lax.fori_loop(0, NS, red, 0)

        @plsc.parallel_loop(0, SL // L, unroll=4)
        def din(g):
            sl = pl.ds(g * L, L)
            d = acc_b[sl]
            acc_b[sl] = jnp.where(d > 0.0, _rsqrt_newton(d), 0.0)

        pltpu.sync_copy(acc_b, dinv_sh.at[pl.ds(base, SL)])
        plsc.subcore_barrier()

        pltpu.sync_copy(dinv_sh, dinv_l)
        ebase = wid * EPW
        pltpu.sync_copy(rc_hbm.at[pl.ds(ebase, EPW)], rc_b.at[pl.ds(0, EPW)])
        pltpu.sync_copy(w_hbm.at[pl.ds(ebase, EPW)], w_b.at[pl.ds(0, EPW)])

        li = lax.iota(jnp.int32, L)
        patt = li * ((1 << SH) + 1)

        @plsc.parallel_loop(0, SLOTW // L, unroll=8)
        def initslots(i):
            sl = pl.ds(i * L, L)
            rcs_loc[sl] = patt
            norm_loc[sl] = jnp.zeros((L,), jnp.float32)

        cnt_b[pl.ds(0, L)] = jnp.zeros((L,), jnp.int32)
        ones = jnp.ones((L,), jnp.int32)

        def nrm(g, ov):
            sl = pl.ds(g * L, L)
            rc = rc_b[sl]
            bnk = lax.shift_right_logical(rc, SH) & (L - 1)
            dr = plsc.load_gather(dinv_l, [lax.shift_right_logical(rc, SH)])
            dc = plsc.load_gather(dinv_l, [rc & MASK])
            nv = (-w_b[sl]) * dr * dc
            ordinal = jnp.zeros((L,), jnp.int32)
            for k in range(1, L):
                prev = bnk.at[jnp.maximum(li - k, 0)].get(
                    mode="promise_in_bounds")
                ordinal = ordinal + jnp.where(
                    (prev == bnk) & (li >= k), 1, 0)
            rankv = plsc.load_gather(cnt_b, [bnk]) + ordinal
            plsc.addupdate_scatter(cnt_b, [bnk], ones)
            d = lax.shift_left(rankv, 4) | bnk
            m = rankv < BL
            plsc.store_scatter(rcs_loc, [d], rc, mask=m)
            plsc.store_scatter(norm_loc, [d], nv, mask=m)
            w_b[sl] = jnp.where(m, 0.0, nv)
            return ov + jnp.where(m, 0, 1)
        ov = lax.fori_loop(0, EPW // L, nrm, jnp.zeros((L,), jnp.int32))

        pltpu.sync_copy(rcs_loc, rcs_hbm.at[pl.ds(wid * SLOTW, SLOTW)])
        pltpu.sync_copy(norm_loc, norms_hbm.at[pl.ds(wid * SLOTW, SLOTW)])
        pltpu.sync_copy(w_b.at[pl.ds(0, EPW)],
                        normb_hbm.at[pl.ds(ebase, EPW)])
        cnt_b[pl.ds(0, L)] = ov
        pltpu.sync_copy(cnt_b.at[pl.ds(0, L)],
                        flag_hbm.at[pl.ds(wid * L, L)])

    return norm_kernel


def _make_prop_kernel(TOT, E_pad, NP, F, SH):
    FPW = F // NW
    FG = 4
    assert FPW % FG == 0
    CA = 4096
    CB = 4096
    NCH = TOT // CA
    assert TOT % CA == 0 and NCH % 2 == 0
    assert E_pad % CB == 0 and CB <= CA
    MASK = (1 << SH) - 1
    mesh = plsc.VectorSubcoreMesh(
        core_axis_name="c", subcore_axis_name="s",
        num_cores=NC, num_subcores=NS)

    @functools.partial(
        pl.kernel, mesh=mesh,
        compiler_params=pltpu.CompilerParams(needs_layout_passes=False),
        out_type=(jax.ShapeDtypeStruct((F, NP), jnp.float32),
                  jax.ShapeDtypeStruct((F, NP), jnp.float32)),
        scratch_types=(
            [pltpu.VMEM((NP,), jnp.float32) for _ in range(2 * FG)] +
            [pltpu.VMEM((CA,), jnp.int32) for _ in range(2)] +
            [pltpu.VMEM((CA,), jnp.float32) for _ in range(2)] +
            [pltpu.VMEM((NW * L,), jnp.int32)] +
            [pltpu.SemaphoreType.DMA for _ in range(4)]
        ),
    )
    def prop_kernel(xT, rcs_hbm, norms_hbm, rco_hbm, normb_hbm, flag_hbm,
                    t1T, t2T,
                    a0, a1, a2, a3, b0, b1, b2, b3,
                    rc0, rc1, nb0, nb1, fbuf,
                    sr0, sr1, sn0, sn1):
        A = [a0, a1, a2, a3]
        B = [b0, b1, b2, b3]
        rcs, nbs = [rc0, rc1], [nb0, nb1]
        srs, sns = [sr0, sr1], [sn0, sn1]
        c = lax.axis_index("c")
        s = lax.axis_index("s")
        wid = s * NC + c
        f0 = wid * FPW
        pltpu.sync_copy(flag_hbm, fbuf)
        fv = fbuf[pl.ds(0, L)]
        for gi in range(1, NW):
            fv = jnp.maximum(fv, fbuf[pl.ds(gi * L, L)])
        fs = jnp.max(fv)

        def start(ci, slot):
            eb = ci * CA
            pltpu.async_copy(rcs_hbm.at[pl.ds(eb, CA)], rcs[slot], srs[slot])
            pltpu.async_copy(norms_hbm.at[pl.ds(eb, CA)], nbs[slot],
                             sns[slot])

        def wait(slot):
            pltpu.make_async_copy(
                rcs_hbm.at[pl.ds(0, CA)], rcs[slot], srs[slot]).wait()
            pltpu.make_async_copy(
                norms_hbm.at[pl.ds(0, CA)], nbs[slot], sns[slot]).wait()

        def make_grp(rcb, nb, src, dst):
            def grp(g):
                sl = pl.ds(g * L, L)
                rc = rcb[sl]
                nv = nb[sl]
                cc = rc & MASK
                r = lax.shift_right_logical(rc, SH)
                vs = [plsc.load_gather(src[ff], [cc]) * nv
                      for ff in range(FG)]
                for ff in range(FG):
                    plsc.addupdate_scatter(dst[ff], [r], vs[ff])
            return grp

        def edge_sweep(src, dst):
            def process(slot):
                plsc.parallel_loop(0, CA // L, unroll=8)(
                    make_grp(rcs[slot], nbs[slot], src, dst))

            start(0, 0)

            def body2(ci2, _):
                ci = ci2 * 2
                start(ci + 1, 1)
                wait(0)
                process(0)

                @pl.when(ci2 < NCH // 2 - 1)
                def _():
                    start(ci + 2, 0)
                wait(1)
                process(1)
                return 0
            lax.fori_loop(0, NCH // 2, body2, 0)

            @pl.when(fs > 0)
            def _():
                def chunkb(ci, _):
                    eb = ci * CB
                    pltpu.sync_copy(rco_hbm.at[pl.ds(eb, CB)],
                                    rcs[0].at[pl.ds(0, CB)])
                    pltpu.sync_copy(normb_hbm.at[pl.ds(eb, CB)],
                                    nbs[0].at[pl.ds(0, CB)])
                    plsc.parallel_loop(0, CB // L, unroll=8)(
                        make_grp(rcs[0], nbs[0], src, dst))
                    return 0
                lax.fori_loop(0, E_pad // CB, chunkb, 0)

        for fg in range(FPW // FG):
            fbase = f0 + fg * FG
            for ff in range(FG):
                pltpu.sync_copy(xT.at[fbase + ff], A[ff])
                _zero_1d(B[ff], NP)
            edge_sweep(A, B)
            for ff in range(FG):
                pltpu.sync_copy(B[ff], t1T.at[fbase + ff])
            for ff in range(FG):
                _zero_1d(A[ff], NP)
            edge_sweep(B, A)
            for ff in range(FG):
                pltpu.sync_copy(xT.at[fbase + ff], B[ff])
                a, xt = A[ff], B[ff]

                @plsc.parallel_loop(0, NP // L, unroll=8)
                def comb(g):
                    sl = pl.ds(g * L, L)
                    a[sl] = 2.0 * a[sl] - xt[sl]

                pltpu.sync_copy(a, t2T.at[fbase + ff])

    return prop_kernel


def _matmul_relu(xT, t1T, t2T, W, b, N, NP, F_IN, F_OUT, BN=1024):
    def mm_kernel(x_ref, t1_ref, t2_ref, w0, w1, w2, b_ref, o_ref):
        dn = (((0,), (0,)), ((), ()))
        acc = lax.dot_general(x_ref[...], w0[...], dn,
                              preferred_element_type=jnp.float32)
        acc = acc + lax.dot_general(t1_ref[...], w1[...], dn,
                                    preferred_element_type=jnp.float32)
        acc = acc + lax.dot_general(t2_ref[...], w2[...], dn,
                                    preferred_element_type=jnp.float32)
        acc = acc + b_ref[...]
        o_ref[...] = jnp.maximum(acc, 0.0)

    grid = (pl.cdiv(N, BN),)
    fspec = pl.BlockSpec((F_IN, BN), lambda i: (0, i))
    wspec = pl.BlockSpec((F_IN, F_OUT), lambda i: (0, 0))
    return pl.pallas_call(
        mm_kernel,
        grid=grid,
        in_specs=[fspec, fspec, fspec, wspec, wspec, wspec,
                  pl.BlockSpec((1, F_OUT), lambda i: (0, 0))],
        out_specs=pl.BlockSpec((BN, F_OUT), lambda i: (i, 0)),
        out_shape=jax.ShapeDtypeStruct((N, F_OUT), jnp.float32),
    )(xT, t1T, t2T, W[0], W[1], W[2], b.reshape(1, F_OUT))


def kernel(x, edge_index, edge_weight, W, b):
    N, F_IN = x.shape
    F_OUT = W.shape[2]
    E = edge_weight.shape[0]
    i32 = jnp.int32

    E_pad = -(-E // (NW * 128)) * (NW * 128)
    pad = E_pad - E
    SH = max((N - 1).bit_length(), 4)
    assert 2 * SH <= 31
    lanes_pad = (jnp.arange(pad, dtype=i32) % L) * ((1 << SH) + 1)
    rc = jnp.concatenate(
        [edge_index[0] << SH | edge_index[1], lanes_pad])
    w_p = jnp.pad(edge_weight, (0, pad))
    NP = -(-N // (NS * L)) * (NS * L)

    SLOTW = BL * L
    TOT = NW * SLOTW

    rc_s, norm_s, norm_ovf, flag = _make_norm_kernel(E_pad, NP, SH)(rc, w_p)
    xTp = jnp.pad(x.T, ((0, 0), (0, NP - N)))
    t1T, t2T = _make_prop_kernel(TOT, E_pad, NP, F_IN, SH)(
        xTp, rc_s, norm_s, rc, norm_ovf, flag)
    return _matmul_relu(xTp, t1T, t2T, W, b, N, NP, F_IN, F_OUT)


# speedup vs baseline: 3.1934x; 1.0884x over previous
import functools

import jax
import jax.numpy as jnp
from jax import lax
from jax.experimental import pallas as pl
from jax.experimental.pallas import tpu as pltpu
from jax.experimental.pallas import tpu_sc as plsc

NC = 2
NS = 16
L = 16
NW = NC * NS
BL = 448


def _rsqrt_newton(d):
    bits = lax.bitcast_convert_type(d, jnp.int32)
    y = lax.bitcast_convert_type(
        jnp.int32(0x5F3759DF) - lax.shift_right_logical(bits, 1), jnp.float32)
    for _ in range(3):
        y = y * (1.5 - 0.5 * d * y * y)
    return y


def _zero_1d(ref, n):
    @plsc.parallel_loop(0, n // L, unroll=8)
    def z(i):
        ref[pl.ds(i * L, L)] = jnp.zeros((L,), jnp.float32)


def _make_norm_kernel(E_pad, NP, SH):
    EPT = E_pad // NS
    EPW = E_pad // NW
    SLOTW = BL * L
    SL = NP // NS
    MASK = (1 << SH) - 1
    mesh = plsc.VectorSubcoreMesh(
        core_axis_name="c", subcore_axis_name="s",
        num_cores=NC, num_subcores=NS)

    @functools.partial(
        pl.kernel, mesh=mesh,
        compiler_params=pltpu.CompilerParams(needs_layout_passes=False),
        out_type=(jax.ShapeDtypeStruct((NW * SLOTW,), jnp.int32),
                  jax.ShapeDtypeStruct((NW * SLOTW,), jnp.float32),
                  jax.ShapeDtypeStruct((E_pad,), jnp.float32),
                  jax.ShapeDtypeStruct((NW * L,), jnp.int32)),
        scratch_types=[
            pltpu.VMEM((NP,), jnp.float32),
            pltpu.VMEM((NP,), jnp.float32),
            pltpu.VMEM((EPT,), jnp.int32),
            pltpu.VMEM((EPT,), jnp.float32),
            pltpu.VMEM((128,), jnp.int32),
            pltpu.VMEM((SLOTW,), jnp.int32),
            pltpu.VMEM((SLOTW,), jnp.float32),
            pltpu.VMEM((SL,), jnp.float32),
            pltpu.VMEM((SL,), jnp.float32),
            pltpu.VMEM_SHARED((NS, NP), jnp.float32),
            pltpu.VMEM_SHARED((NP,), jnp.float32),
        ],
    )
    def norm_kernel(rc_hbm, w_hbm,
                    rcs_hbm, norms_hbm, normb_hbm, flag_hbm,
                    deg_l, dinv_l, rc_b, w_b, cnt_b, rcs_loc, norm_loc,
                    tmp_b, acc_b, deg_sh, dinv_sh):
        c = lax.axis_index("c")
        s = lax.axis_index("s")
        wid = s * NC + c

        _zero_1d(deg_l, NP)
        pltpu.sync_copy(rc_hbm.at[pl.ds(s * EPT, EPT)], rc_b)
        pltpu.sync_copy(w_hbm.at[pl.ds(s * EPT, EPT)], w_b)

        @plsc.parallel_loop(0, EPT // L, unroll=8)
        def acc_deg(g):
            sl = pl.ds(g * L, L)
            r = lax.shift_right_logical(rc_b[sl], SH)
            plsc.addupdate_scatter(deg_l, [r], w_b[sl])

        pltpu.sync_copy(deg_l, deg_sh.at[s])
        plsc.subcore_barrier()

        base = s * SL
        _zero_1d(acc_b, SL)

        def red(j, _):
            pltpu.sync_copy(deg_sh.at[j, pl.ds(base, SL)], tmp_b)

            @plsc.parallel_loop(0, SL // L, unroll=8)
            def addg(g):
                sl = pl.ds(g * L, L)
                acc_b[sl] = acc_b[sl] + tmp_b[sl]
            return 0
        lax.fori_loop(0, NS, red, 0)

        @plsc.parallel_loop(0, SL // L, unroll=4)
        def din(g):
            sl = pl.ds(g * L, L)
            d = acc_b[sl]
            acc_b[sl] = jnp.where(d > 0.0, _rsqrt_newton(d), 0.0)

        pltpu.sync_copy(acc_b, dinv_sh.at[pl.ds(base, SL)])
        plsc.subcore_barrier()

        pltpu.sync_copy(dinv_sh, dinv_l)
        ebase = wid * EPW
        pltpu.sync_copy(rc_hbm.at[pl.ds(ebase, EPW)], rc_b.at[pl.ds(0, EPW)])
        pltpu.sync_copy(w_hbm.at[pl.ds(ebase, EPW)], w_b.at[pl.ds(0, EPW)])

        li = lax.iota(jnp.int32, L)
        patt = li * ((1 << SH) + 1)

        @plsc.parallel_loop(0, SLOTW // L, unroll=8)
        def initslots(i):
            sl = pl.ds(i * L, L)
            rcs_loc[sl] = patt
            norm_loc[sl] = jnp.zeros((L,), jnp.float32)

        cnt_b[pl.ds(0, L)] = jnp.zeros((L,), jnp.int32)
        ones = jnp.ones((L,), jnp.int32)

        def nrm(g, ov):
            sl = pl.ds(g * L, L)
            rc = rc_b[sl]
            bnk = lax.shift_right_logical(rc, SH) & (L - 1)
            dr = plsc.load_gather(dinv_l, [lax.shift_right_logical(rc, SH)])
            dc = plsc.load_gather(dinv_l, [rc & MASK])
            nv = (-w_b[sl]) * dr * dc
            ordinal = jnp.zeros((L,), jnp.int32)
            for k in range(1, L):
                prev = bnk.at[jnp.maximum(li - k, 0)].get(
                    mode="promise_in_bounds")
                ordinal = ordinal + jnp.where(
                    (prev == bnk) & (li >= k), 1, 0)
            rankv = plsc.load_gather(cnt_b, [bnk]) + ordinal
            plsc.addupdate_scatter(cnt_b, [bnk], ones)
            d = lax.shift_left(rankv, 4) | bnk
            m = rankv < BL
            plsc.store_scatter(rcs_loc, [d], rc, mask=m)
            plsc.store_scatter(norm_loc, [d], nv, mask=m)
            w_b[sl] = jnp.where(m, 0.0, nv)
            return ov + jnp.where(m, 0, 1)
        ov = lax.fori_loop(0, EPW // L, nrm, jnp.zeros((L,), jnp.int32))

        pltpu.sync_copy(rcs_loc, rcs_hbm.at[pl.ds(wid * SLOTW, SLOTW)])
        pltpu.sync_copy(norm_loc, norms_hbm.at[pl.ds(wid * SLOTW, SLOTW)])
        pltpu.sync_copy(w_b.at[pl.ds(0, EPW)],
                        normb_hbm.at[pl.ds(ebase, EPW)])
        cnt_b[pl.ds(0, L)] = ov
        pltpu.sync_copy(cnt_b.at[pl.ds(0, L)],
                        flag_hbm.at[pl.ds(wid * L, L)])

    return norm_kernel


def _make_prop_kernel(TOT, E_pad, NP, F, SH):
    FPW = F // NW
    FG = 4
    assert FPW % FG == 0
    CA = 4096
    CB = 4096
    NCH = TOT // CA
    assert TOT % CA == 0 and NCH % 2 == 0
    assert E_pad % CB == 0 and CB <= CA
    MASK = (1 << SH) - 1
    mesh = plsc.VectorSubcoreMesh(
        core_axis_name="c", subcore_axis_name="s",
        num_cores=NC, num_subcores=NS)

    @functools.partial(
        pl.kernel, mesh=mesh,
        compiler_params=pltpu.CompilerParams(needs_layout_passes=False),
        out_type=(jax.ShapeDtypeStruct((F, NP), jnp.float32),
                  jax.ShapeDtypeStruct((F, NP), jnp.float32)),
        scratch_types=(
            [pltpu.VMEM((NP,), jnp.float32) for _ in range(2 * FG)] +
            [pltpu.VMEM((CA,), jnp.int32) for _ in range(2)] +
            [pltpu.VMEM((CA,), jnp.float32) for _ in range(2)] +
            [pltpu.VMEM((NW * L,), jnp.int32)] +
            [pltpu.SemaphoreType.DMA for _ in range(4)]
        ),
    )
    def prop_kernel(xT, rcs_hbm, norms_hbm, rco_hbm, normb_hbm, flag_hbm,
                    t1T, t2T,
                    a0, a1, a2, a3, b0, b1, b2, b3,
                    rc0, rc1, nb0, nb1, fbuf,
                    sr0, sr1, sn0, sn1):
        A = [a0, a1, a2, a3]
        B = [b0, b1, b2, b3]
        rcs, nbs = [rc0, rc1], [nb0, nb1]
        srs, sns = [sr0, sr1], [sn0, sn1]
        c = lax.axis_index("c")
        s = lax.axis_index("s")
        wid = s * NC + c
        f0 = wid * FPW
        pltpu.sync_copy(flag_hbm, fbuf)
        fv = fbuf[pl.ds(0, L)]
        for gi in range(1, NW):
            fv = jnp.maximum(fv, fbuf[pl.ds(gi * L, L)])
        fs = jnp.max(fv)

        def start(ci, slot):
            eb = ci * CA
            pltpu.async_copy(rcs_hbm.at[pl.ds(eb, CA)], rcs[slot], srs[slot])
            pltpu.async_copy(norms_hbm.at[pl.ds(eb, CA)], nbs[slot],
                             sns[slot])

        def wait(slot):
            pltpu.make_async_copy(
                rcs_hbm.at[pl.ds(0, CA)], rcs[slot], srs[slot]).wait()
            pltpu.make_async_copy(
                norms_hbm.at[pl.ds(0, CA)], nbs[slot], sns[slot]).wait()

        def make_grp(rcb, nb, src, dst):
            def grp(g):
                sl = pl.ds(g * L, L)
                rc = rcb[sl]
                nv = nb[sl]
                cc = rc & MASK
                r = lax.shift_right_logical(rc, SH)
                vs = [plsc.load_gather(src[ff], [cc]) * nv
                      for ff in range(FG)]
                for ff in range(FG):
                    plsc.addupdate_scatter(dst[ff], [r], vs[ff])
            return grp

        def edge_sweep(src, dst):
            def process(slot):
                plsc.parallel_loop(0, CA // L, unroll=8)(
                    make_grp(rcs[slot], nbs[slot], src, dst))

            start(0, 0)

            def body2(ci2, _):
                ci = ci2 * 2
                start(ci + 1, 1)
                wait(0)
                process(0)

                @pl.when(ci2 < NCH // 2 - 1)
                def _():
                    start(ci + 2, 0)
                wait(1)
                process(1)
                return 0
            lax.fori_loop(0, NCH // 2, body2, 0)

            @pl.when(fs > 0)
            def _():
                def chunkb(ci, _):
                    eb = ci * CB
                    pltpu.sync_copy(rco_hbm.at[pl.ds(eb, CB)],
                                    rcs[0].at[pl.ds(0, CB)])
                    pltpu.sync_copy(normb_hbm.at[pl.ds(eb, CB)],
                                    nbs[0].at[pl.ds(0, CB)])
                    plsc.parallel_loop(0, CB // L, unroll=8)(
                        make_grp(rcs[0], nbs[0], src, dst))
                    return 0
                lax.fori_loop(0, E_pad // CB, chunkb, 0)

        for fg in range(FPW // FG):
            fbase = f0 + fg * FG
            for ff in range(FG):
                pltpu.sync_copy(xT.at[fbase + ff], A[ff])
                _zero_1d(B[ff], NP)
            edge_sweep(A, B)
            for ff in range(FG):
                pltpu.sync_copy(B[ff], t1T.at[fbase + ff])
            for ff in range(FG):
                _zero_1d(A[ff], NP)
            edge_sweep(B, A)
            for ff in range(FG):
                pltpu.sync_copy(xT.at[fbase + ff], B[ff])
                a, xt = A[ff], B[ff]

                @plsc.parallel_loop(0, NP // L, unroll=8)
                def comb(g):
                    sl = pl.ds(g * L, L)
                    a[sl] = 2.0 * a[sl] - xt[sl]

                pltpu.sync_copy(a, t2T.at[fbase + ff])

    return prop_kernel


def _matmul_relu(xT, t1T, t2T, W, b, N, NP, F_IN, F_OUT, BN=1024):
    def mm_kernel(x_ref, t1_ref, t2_ref, w0, w1, w2, b_ref, o_ref):
        dn = (((0,), (0,)), ((), ()))
        acc = lax.dot_general(x_ref[...], w0[...], dn,
                              preferred_element_type=jnp.float32)
        acc = acc + lax.dot_general(t1_ref[...], w1[...], dn,
                                    preferred_element_type=jnp.float32)
        acc = acc + lax.dot_general(t2_ref[...], w2[...], dn,
                                    preferred_element_type=jnp.float32)
        acc = acc + b_ref[...]
        o_ref[...] = jnp.maximum(acc, 0.0)

    grid = (pl.cdiv(N, BN),)
    fspec = pl.BlockSpec((F_IN, BN), lambda i: (0, i))
    wspec = pl.BlockSpec((F_IN, F_OUT), lambda i: (0, 0))
    return pl.pallas_call(
        mm_kernel,
        grid=grid,
        in_specs=[fspec, fspec, fspec, wspec, wspec, wspec,
                  pl.BlockSpec((1, F_OUT), lambda i: (0, 0))],
        out_specs=pl.BlockSpec((BN, F_OUT), lambda i: (i, 0)),
        out_shape=jax.ShapeDtypeStruct((N, F_OUT), jnp.float32),
    )(xT, t1T, t2T, W[0], W[1], W[2], b.reshape(1, F_OUT))


def kernel(x, edge_index, edge_weight, W, b):
    N, F_IN = x.shape
    F_OUT = W.shape[2]
    E = edge_weight.shape[0]
    i32 = jnp.int32

    E_pad = -(-E // (NW * 128)) * (NW * 128)
    pad = E_pad - E
    SH = max((N - 1).bit_length(), 4)
    assert 2 * SH <= 31
    lanes_pad = (jnp.arange(pad, dtype=i32) % L) * ((1 << SH) + 1)
    rc = jnp.concatenate(
        [edge_index[0] << SH | edge_index[1], lanes_pad])
    w_p = jnp.pad(edge_weight, (0, pad))
    NP = -(-N // (NS * L)) * (NS * L)

    SLOTW = BL * L
    TOT = NW * SLOTW

    rc_s, norm_s, norm_ovf, flag = _make_norm_kernel(E_pad, NP, SH)(rc, w_p)
    xTp = jnp.pad(x.T, ((0, 0), (0, NP - N)))
    t1T, t2T = _make_prop_kernel(TOT, E_pad, NP, F_IN, SH)(
        xTp, rc_s, norm_s, rc, norm_ovf, flag)
    return _matmul_relu(xTp, t1T, t2T, W, b, N, NP, F_IN, F_OUT)
